# baseline (speedup 1.0000x reference)
import functools

import jax
import jax.numpy as jnp
from jax import lax
from jax.experimental import pallas as pl
from jax.experimental.pallas import tpu as pltpu
from jax.experimental.pallas import tpu_sc as plsc

N_ROWS = 16906
DIM = 512

_info = plsc.get_sparse_core_info()
_NC, _NS = _info.num_cores, _info.num_subcores
_CHUNK = 8448
_REM = N_ROWS - _NC * _CHUNK
_CROWS = 1056
_NCH = _CHUNK // _CROWS
_NBUF = 3

_mesh = plsc.VectorSubcoreMesh(core_axis_name="c", subcore_axis_name="s")


@functools.partial(
    pl.kernel,
    mesh=_mesh,
    out_type=jax.ShapeDtypeStruct((N_ROWS, DIM), jnp.float32),
    scratch_types=[
        pltpu.VMEM_SHARED((_NBUF, _CROWS, DIM), jnp.float32),
        pltpu.VMEM((1, DIM), jnp.float32),
        pltpu.SemaphoreType.DMA,
        pltpu.SemaphoreType.DMA,
        pltpu.SemaphoreType.DMA,
        pltpu.SemaphoreType.DMA,
        pltpu.SemaphoreType.DMA,
        pltpu.SemaphoreType.DMA,
    ],
)
def _slice_copy(table_hbm, out_hbm, buf, rowbuf, si0, si1, si2, so0, so1, so2):
    cid = lax.axis_index("c")
    sid = lax.axis_index("s")
    in_sems = (si0, si1, si2)
    out_sems = (so0, so1, so2)

    @pl.when(sid == 0)
    def _main():
        base = cid * _CHUNK
        in_d = [None] * _NCH
        out_d = [None] * _NCH

        def start_in(i):
            off = base + i * _CROWS
            in_d[i] = pltpu.async_copy(
                table_hbm.at[pl.ds(off, _CROWS)], buf.at[i % _NBUF],
                in_sems[i % _NBUF])

        def start_out(i):
            off = base + i * _CROWS
            out_d[i] = pltpu.async_copy(
                buf.at[i % _NBUF], out_hbm.at[pl.ds(off, _CROWS)],
                out_sems[i % _NBUF])

        for j in range(_NBUF - 1):
            start_in(j)
        for i in range(_NCH):
            j = i + _NBUF - 1
            if j < _NCH:
                if i >= 1:
                    out_d[i - 1].wait()
                start_in(j)
            in_d[i].wait()
            start_out(i)
        for i in range(max(0, _NCH - _NBUF), _NCH):
            out_d[i].wait()

    @pl.when((cid == 1) & (sid < _REM))
    def _tail():
        r = _NC * _CHUNK + sid
        pltpu.sync_copy(table_hbm.at[pl.ds(r, 1)], rowbuf)
        pltpu.sync_copy(rowbuf, out_hbm.at[pl.ds(r, 1)])


def kernel(x, table):
    del x
    return _slice_copy(table)


# speedup vs baseline: 1.0925x; 1.0925x over previous
import functools

import jax
import jax.numpy as jnp
from jax import lax
from jax.experimental import pallas as pl
from jax.experimental.pallas import tpu as pltpu
from jax.experimental.pallas import tpu_sc as plsc

N_ROWS = 16906
DIM = 512

_info = plsc.get_sparse_core_info()
_NC, _NS = _info.num_cores, _info.num_subcores

_SP_BLOCKS = (1080, 1056, 1056, 1056)
_SP_CHUNK = sum(_SP_BLOCKS)
_SP_TOTAL = _NC * _SP_CHUNK
_SP_BUF_ROWS = max(_SP_BLOCKS)
_SP_NBUF = 2

_ST_WORKERS = (_NS - 1) * _NC
_ST_CHUNK = 280
_ST_CROWS = 40
_ST_NCH = _ST_CHUNK // _ST_CROWS
_ST_TOTAL = _ST_WORKERS * _ST_CHUNK
_ST_NBUF = 3

_TAIL_BASE = _SP_TOTAL + _ST_TOTAL
_REM = N_ROWS - _TAIL_BASE

_mesh = plsc.VectorSubcoreMesh(core_axis_name="c", subcore_axis_name="s")


def _pipeline(blocks, nbuf, base, table_hbm, out_hbm, buf, in_sems, out_sems):
    nch = len(blocks)
    offs = []
    o = base
    for b in blocks:
        offs.append(o)
        o = o + b

    in_d = [None] * nch
    out_d = [None] * nch

    def start_in(i):
        in_d[i] = pltpu.async_copy(
            table_hbm.at[pl.ds(offs[i], blocks[i])],
            buf.at[i % nbuf, pl.ds(0, blocks[i])],
            in_sems[i % nbuf])

    def start_out(i):
        out_d[i] = pltpu.async_copy(
            buf.at[i % nbuf, pl.ds(0, blocks[i])],
            out_hbm.at[pl.ds(offs[i], blocks[i])],
            out_sems[i % nbuf])

    for j in range(min(nbuf - 1, nch)):
        start_in(j)
    for i in range(nch):
        j = i + nbuf - 1
        if j < nch:
            if i >= 1:
                out_d[i - 1].wait()
            start_in(j)
        in_d[i].wait()
        start_out(i)
    for i in range(max(0, nch - nbuf), nch):
        out_d[i].wait()


@functools.partial(
    pl.kernel,
    mesh=_mesh,
    out_type=jax.ShapeDtypeStruct((N_ROWS, DIM), jnp.float32),
    scratch_types=[
        pltpu.VMEM_SHARED((_SP_NBUF, _SP_BUF_ROWS, DIM), jnp.float32),
        pltpu.VMEM((_ST_NBUF, _ST_CROWS, DIM), jnp.float32),
        pltpu.SemaphoreType.DMA,
        pltpu.SemaphoreType.DMA,
        pltpu.SemaphoreType.DMA,
        pltpu.SemaphoreType.DMA,
        pltpu.SemaphoreType.DMA,
        pltpu.SemaphoreType.DMA,
    ],
)
def _slice_copy(table_hbm, out_hbm, spbuf, stbuf,
                si0, si1, si2, so0, so1, so2):
    cid = lax.axis_index("c")
    sid = lax.axis_index("s")
    in_sems = (si0, si1, si2)
    out_sems = (so0, so1, so2)

    @pl.when(sid == 0)
    def _spmem_main():
        _pipeline(_SP_BLOCKS, _SP_NBUF, cid * _SP_CHUNK,
                  table_hbm, out_hbm, spbuf, in_sems, out_sems)

    @pl.when(sid > 0)
    def _stream_main():
        k = (sid - 1) * _NC + cid
        base = _SP_TOTAL + k * _ST_CHUNK
        _pipeline((_ST_CROWS,) * _ST_NCH, _ST_NBUF, base,
                  table_hbm, out_hbm, stbuf, in_sems, out_sems)

        @pl.when(k < _REM)
        def _tail():
            r = _TAIL_BASE + k
            row = stbuf.at[0, pl.ds(0, 1)]
            pltpu.sync_copy(table_hbm.at[pl.ds(r, 1)], row)
            pltpu.sync_copy(row, out_hbm.at[pl.ds(r, 1)])


def kernel(x, table):
    del x
    return _slice_copy(table)
